# baseline (speedup 1.0000x reference)
import functools

import jax
import jax.numpy as jnp
from jax import lax
from jax.experimental import pallas as pl
from jax.experimental.pallas import tpu as pltpu
from jax.experimental.pallas import tpu_sc as plsc

VOCAB = 1000000
D = 64
B = 16384
CTX = 20
NEG = 20
NR = D // 16

NC = 2
NS = 16
NW = NC * NS
BPW = B // NW
CB = 16
NCHUNK = BPW // CB
IDS_PER_CHUNK = CB * CTX
QROWS = 80
NQ = IDS_PER_CHUNK // QROWS
PSTRIDE = 17
KSLOT = CB * PSTRIDE
NK = NEG + 1


def _sc_body(ctx_ids2, neg_ids2, tgt_ids2, in_embed, out_embed,
             pos_out, neg_out,
             ctx_idx, neg_idx, tgt_idx, rows, pbuf, pos_stage, neg_stage,
             rowsem, outsem):
    c = lax.axis_index("c")
    s = lax.axis_index("s")
    wid = s * NC + c
    iota = lax.iota(jnp.int32, 16)
    iota17 = iota * PSTRIDE
    iota20 = iota * NEG

    pltpu.sync_copy(ctx_ids2.at[pl.ds(wid * (BPW * CTX // QROWS),
                                      BPW * CTX // QROWS)], ctx_idx)
    pltpu.sync_copy(neg_ids2.at[pl.ds(wid * (BPW * CTX // QROWS),
                                      BPW * CTX // QROWS)], neg_idx)
    pltpu.sync_copy(tgt_ids2.at[pl.ds(wid * NCHUNK, NCHUNK)], tgt_idx)

    def issue_rows(i):
        p = lax.rem(i, 2)
        r0 = i * NQ
        for q in range(NQ):
            pltpu.async_copy(
                in_embed.at[ctx_idx.at[r0 + q]],
                rows.at[p, pl.ds(q * QROWS, QROWS)], rowsem.at[p])
            pltpu.async_copy(
                out_embed.at[neg_idx.at[r0 + q]],
                rows.at[p, pl.ds(IDS_PER_CHUNK + q * QROWS, QROWS)],
                rowsem.at[p])
        pltpu.async_copy(out_embed.at[tgt_idx.at[i]],
                         rows.at[p, pl.ds(2 * IDS_PER_CHUNK, CB)],
                         rowsem.at[p])

    def drain_rows(i):
        p = lax.rem(i, 2)
        for q in range(NQ):
            pltpu.make_async_copy(
                in_embed.at[ctx_idx.at[0]],
                rows.at[p, pl.ds(q * QROWS, QROWS)], rowsem.at[p]).wait()
            pltpu.make_async_copy(
                out_embed.at[neg_idx.at[0]],
                rows.at[p, pl.ds(IDS_PER_CHUNK + q * QROWS, QROWS)],
                rowsem.at[p]).wait()
        pltpu.make_async_copy(out_embed.at[tgt_idx.at[0]],
                              rows.at[p, pl.ds(2 * IDS_PER_CHUNK, CB)],
                              rowsem.at[p]).wait()

    issue_rows(0)

    def chunk_body(i, carry_unused):
        p = lax.rem(i, 2)
        b0 = wid * BPW + i * CB

        @pl.when(i < NCHUNK - 1)
        def _():
            issue_rows(i + 1)

        drain_rows(i)

        def b_body(b, carry_unused2):
            base = b * CTX
            vc = [rows[p, base, pl.ds(r * 16, 16)] for r in range(NR)]
            for j in range(1, CTX):
                for r in range(NR):
                    vc[r] = vc[r] + rows[p, base + j, pl.ds(r * 16, 16)]
            inv = jnp.float32(1.0 / CTX)
            vc = [v * inv for v in vc]

            def dot_partial(row):
                t = [vc[r] * rows[p, row, pl.ds(r * 16, 16)]
                     for r in range(NR)]
                return (t[0] + t[1]) + (t[2] + t[3])

            slot = b * PSTRIDE
            pbuf[pl.ds(NEG * KSLOT + slot, 16)] = dot_partial(
                2 * IDS_PER_CHUNK + b)
            for k in range(NEG):
                pbuf[pl.ds(k * KSLOT + slot, 16)] = dot_partial(
                    IDS_PER_CHUNK + base + k)
            return carry_unused2

        lax.fori_loop(0, CB, b_body, 0, unroll=False)

        @pl.when(i >= 2)
        def _():
            pltpu.make_async_copy(pos_stage.at[p],
                                  pos_out.at[pl.ds(0, CB)], outsem.at[p]
                                  ).wait()
            pltpu.make_async_copy(neg_stage.at[p],
                                  neg_out.at[pl.ds(0, CB * NEG)],
                                  outsem.at[p]).wait()

        def treduce(k):
            vals = [plsc.load_gather(pbuf, [iota17 + (k * KSLOT + j)])
                    for j in range(16)]
            while len(vals) > 1:
                vals = [vals[2 * m] + vals[2 * m + 1]
                        for m in range(len(vals) // 2)]
            return vals[0]

        pos_stage[p] = treduce(NEG)
        for k in range(NEG):
            plsc.store_scatter(neg_stage.at[p], [iota20 + k], treduce(k))

        pltpu.async_copy(pos_stage.at[p], pos_out.at[pl.ds(b0, CB)],
                         outsem.at[p])
        pltpu.async_copy(neg_stage.at[p],
                         neg_out.at[pl.ds(b0 * NEG, CB * NEG)],
                         outsem.at[p])
        return carry_unused

    lax.fori_loop(0, NCHUNK, chunk_body, 0)

    for p in range(2):
        pltpu.make_async_copy(pos_stage.at[p], pos_out.at[pl.ds(0, CB)],
                              outsem.at[p]).wait()
        pltpu.make_async_copy(neg_stage.at[p],
                              neg_out.at[pl.ds(0, CB * NEG)],
                              outsem.at[p]).wait()


def _tp_body(src_ref, dst_ref):
    x = src_ref[...]
    r = jax.lax.broadcasted_iota(jnp.int32, (D, D), 0)
    c = jax.lax.broadcasted_iota(jnp.int32, (D, D), 1)
    iden = (r == c).astype(jnp.float32)
    dst_ref[...] = jax.lax.dot_general(
        x, iden, (((0,), (0,)), ((), ())),
        preferred_element_type=jnp.float32,
        precision=jax.lax.Precision.HIGHEST)


_TP_PARAMS = pltpu.CompilerParams(fuse_transposed_lhs_in_matmul=True)


def _transpose_table(table_t):
    s = 2048
    return pl.pallas_call(
        _tp_body,
        grid=((VOCAB + s - 1) // s,),
        in_specs=[pl.BlockSpec((D, s), lambda i: (0, i))],
        out_specs=pl.BlockSpec((s, D), lambda i: (i, 0)),
        out_shape=jax.ShapeDtypeStruct((VOCAB, D), jnp.float32),
        compiler_params=_TP_PARAMS,
    )(table_t)


@jax.jit
def _sc_call(ctx_ids2, neg_ids2, tgt_ids2, in_embed, out_embed):
    mesh = plsc.VectorSubcoreMesh(core_axis_name="c", subcore_axis_name="s")
    f = functools.partial(
        pl.kernel,
        out_type=[
            jax.ShapeDtypeStruct((B,), jnp.float32),
            jax.ShapeDtypeStruct((B * NEG,), jnp.float32),
        ],
        mesh=mesh,
        compiler_params=pltpu.CompilerParams(
            needs_layout_passes=False, use_tc_tiling_on_sc=False),
        scratch_types=[
            pltpu.VMEM((BPW * CTX // QROWS, QROWS), jnp.int32),
            pltpu.VMEM((BPW * CTX // QROWS, QROWS), jnp.int32),
            pltpu.VMEM((NCHUNK, CB), jnp.int32),
            pltpu.VMEM((2, 2 * IDS_PER_CHUNK + CB, D), jnp.float32),
            pltpu.VMEM((NK * KSLOT,), jnp.float32),
            pltpu.VMEM((2, CB), jnp.float32),
            pltpu.VMEM((2, CB * NEG), jnp.float32),
            pltpu.SemaphoreType.DMA((2,)),
            pltpu.SemaphoreType.DMA((2,)),
        ],
    )(_sc_body)
    return f(ctx_ids2, neg_ids2, tgt_ids2, in_embed, out_embed)


def kernel(context_ids, target_ids, neg_ids, in_embed, out_embed):
    ctx2 = jnp.asarray(context_ids, jnp.int32).reshape(B * CTX // QROWS, QROWS)
    neg2 = jnp.asarray(neg_ids, jnp.int32).reshape(B * NEG // QROWS, QROWS)
    tgt2 = jnp.asarray(target_ids, jnp.int32).reshape(B // CB, CB)
    in2 = _transpose_table(in_embed.T)
    out2 = _transpose_table(out_embed.T)
    pos, neg_flat = _sc_call(ctx2, neg2, tgt2, in2, out2)
    return pos, neg_flat.reshape(B, NEG)


# speedup vs baseline: 1.1342x; 1.1342x over previous
import functools

import jax
import jax.numpy as jnp
from jax import lax
from jax.experimental import pallas as pl
from jax.experimental.pallas import tpu as pltpu
from jax.experimental.pallas import tpu_sc as plsc

VOCAB = 1000000
D = 64
B = 16384
CTX = 20
NEG = 20
NR = D // 16

NC = 2
NS = 16
NW = NC * NS
BPW = B // NW
CB = 16
NCHUNK = BPW // CB
IDS_PER_CHUNK = CB * CTX
QROWS = 80
NQ = IDS_PER_CHUNK // QROWS
PSTRIDE = 17
KSLOT = CB * PSTRIDE
NK = NEG + 1


def _sc_body(ctx_ids2, neg_ids2, tgt_ids2, in_embed, out_embed,
             pos_out, neg_out,
             ctx_idx, neg_idx, tgt_idx, rows, pbuf, pos_stage, neg_stage,
             rowsem, outsem):
    c = lax.axis_index("c")
    s = lax.axis_index("s")
    wid = s * NC + c
    iota = lax.iota(jnp.int32, 16)
    iota17 = iota * PSTRIDE
    iota20 = iota * NEG

    pltpu.sync_copy(ctx_ids2.at[pl.ds(wid * (BPW * CTX // QROWS),
                                      BPW * CTX // QROWS)], ctx_idx)
    pltpu.sync_copy(neg_ids2.at[pl.ds(wid * (BPW * CTX // QROWS),
                                      BPW * CTX // QROWS)], neg_idx)
    pltpu.sync_copy(tgt_ids2.at[pl.ds(wid * NCHUNK, NCHUNK)], tgt_idx)

    def issue_rows(i):
        p = lax.rem(i, 2)
        r0 = i * NQ
        for q in range(NQ):
            pltpu.async_copy(
                in_embed.at[ctx_idx.at[r0 + q]],
                rows.at[p, pl.ds(q * QROWS, QROWS)], rowsem.at[p])
            pltpu.async_copy(
                out_embed.at[neg_idx.at[r0 + q]],
                rows.at[p, pl.ds(IDS_PER_CHUNK + q * QROWS, QROWS)],
                rowsem.at[p])
        pltpu.async_copy(out_embed.at[tgt_idx.at[i]],
                         rows.at[p, pl.ds(2 * IDS_PER_CHUNK, CB)],
                         rowsem.at[p])

    def drain_rows(i):
        p = lax.rem(i, 2)
        for q in range(NQ):
            pltpu.make_async_copy(
                in_embed.at[ctx_idx.at[0]],
                rows.at[p, pl.ds(q * QROWS, QROWS)], rowsem.at[p]).wait()
            pltpu.make_async_copy(
                out_embed.at[neg_idx.at[0]],
                rows.at[p, pl.ds(IDS_PER_CHUNK + q * QROWS, QROWS)],
                rowsem.at[p]).wait()
        pltpu.make_async_copy(out_embed.at[tgt_idx.at[0]],
                              rows.at[p, pl.ds(2 * IDS_PER_CHUNK, CB)],
                              rowsem.at[p]).wait()

    issue_rows(0)

    def chunk_body(i, carry_unused):
        p = lax.rem(i, 2)
        b0 = wid * BPW + i * CB

        @pl.when(i < NCHUNK - 1)
        def _():
            issue_rows(i + 1)

        drain_rows(i)

        def b_body(b, carry_unused2):
            base = b * CTX
            vc = [rows[p, base, pl.ds(r * 16, 16)] for r in range(NR)]
            for j in range(1, CTX):
                for r in range(NR):
                    vc[r] = vc[r] + rows[p, base + j, pl.ds(r * 16, 16)]
            inv = jnp.float32(1.0 / CTX)
            vc = [v * inv for v in vc]

            def dot_partial(row):
                t = [vc[r] * rows[p, row, pl.ds(r * 16, 16)]
                     for r in range(NR)]
                return (t[0] + t[1]) + (t[2] + t[3])

            slot = b * PSTRIDE
            pbuf[pl.ds(NEG * KSLOT + slot, 16)] = dot_partial(
                2 * IDS_PER_CHUNK + b)
            for k in range(NEG):
                pbuf[pl.ds(k * KSLOT + slot, 16)] = dot_partial(
                    IDS_PER_CHUNK + base + k)
            return carry_unused2

        lax.fori_loop(0, CB, b_body, 0, unroll=False)

        @pl.when(i >= 2)
        def _():
            pltpu.make_async_copy(pos_stage.at[p],
                                  pos_out.at[pl.ds(0, CB)], outsem.at[p]
                                  ).wait()
            pltpu.make_async_copy(neg_stage.at[p],
                                  neg_out.at[pl.ds(0, CB * NEG)],
                                  outsem.at[p]).wait()

        def treduce(k):
            vals = [plsc.load_gather(pbuf, [iota17 + (k * KSLOT + j)])
                    for j in range(16)]
            while len(vals) > 1:
                vals = [vals[2 * m] + vals[2 * m + 1]
                        for m in range(len(vals) // 2)]
            return vals[0]

        pos_stage[p] = treduce(NEG)
        for k in range(NEG):
            plsc.store_scatter(neg_stage.at[p], [iota20 + k], treduce(k))

        pltpu.async_copy(pos_stage.at[p], pos_out.at[pl.ds(b0, CB)],
                         outsem.at[p])
        pltpu.async_copy(neg_stage.at[p],
                         neg_out.at[pl.ds(b0 * NEG, CB * NEG)],
                         outsem.at[p])
        return carry_unused

    lax.fori_loop(0, NCHUNK, chunk_body, 0)

    for p in range(2):
        pltpu.make_async_copy(pos_stage.at[p], pos_out.at[pl.ds(0, CB)],
                              outsem.at[p]).wait()
        pltpu.make_async_copy(neg_stage.at[p],
                              neg_out.at[pl.ds(0, CB * NEG)],
                              outsem.at[p]).wait()


def _tp_body(src_ref, dst_ref):
    x = src_ref[...]
    r = jax.lax.broadcasted_iota(jnp.int32, (D, D), 0)
    c = jax.lax.broadcasted_iota(jnp.int32, (D, D), 1)
    iden = (r == c).astype(jnp.float32)
    dst_ref[...] = jax.lax.dot_general(
        x, iden, (((0,), (0,)), ((), ())),
        preferred_element_type=jnp.float32,
        precision=jax.lax.Precision.DEFAULT)


_TP_PARAMS = pltpu.CompilerParams(fuse_transposed_lhs_in_matmul=True)


def _transpose_table(table_t):
    s = 2048
    return pl.pallas_call(
        _tp_body,
        grid=((VOCAB + s - 1) // s,),
        in_specs=[pl.BlockSpec((D, s), lambda i: (0, i))],
        out_specs=pl.BlockSpec((s, D), lambda i: (i, 0)),
        out_shape=jax.ShapeDtypeStruct((VOCAB, D), jnp.float32),
        compiler_params=_TP_PARAMS,
    )(table_t)


@jax.jit
def _sc_call(ctx_ids2, neg_ids2, tgt_ids2, in_embed, out_embed):
    mesh = plsc.VectorSubcoreMesh(core_axis_name="c", subcore_axis_name="s")
    f = functools.partial(
        pl.kernel,
        out_type=[
            jax.ShapeDtypeStruct((B,), jnp.float32),
            jax.ShapeDtypeStruct((B * NEG,), jnp.float32),
        ],
        mesh=mesh,
        compiler_params=pltpu.CompilerParams(
            needs_layout_passes=False, use_tc_tiling_on_sc=False),
        scratch_types=[
            pltpu.VMEM((BPW * CTX // QROWS, QROWS), jnp.int32),
            pltpu.VMEM((BPW * CTX // QROWS, QROWS), jnp.int32),
            pltpu.VMEM((NCHUNK, CB), jnp.int32),
            pltpu.VMEM((2, 2 * IDS_PER_CHUNK + CB, D), jnp.float32),
            pltpu.VMEM((NK * KSLOT,), jnp.float32),
            pltpu.VMEM((2, CB), jnp.float32),
            pltpu.VMEM((2, CB * NEG), jnp.float32),
            pltpu.SemaphoreType.DMA((2,)),
            pltpu.SemaphoreType.DMA((2,)),
        ],
    )(_sc_body)
    return f(ctx_ids2, neg_ids2, tgt_ids2, in_embed, out_embed)


def kernel(context_ids, target_ids, neg_ids, in_embed, out_embed):
    ctx2 = jnp.asarray(context_ids, jnp.int32).reshape(B * CTX // QROWS, QROWS)
    neg2 = jnp.asarray(neg_ids, jnp.int32).reshape(B * NEG // QROWS, QROWS)
    tgt2 = jnp.asarray(target_ids, jnp.int32).reshape(B // CB, CB)
    in2 = _transpose_table(in_embed.T)
    out2 = _transpose_table(out_embed.T)
    pos, neg_flat = _sc_call(ctx2, neg2, tgt2, in2, out2)
    return pos, neg_flat.reshape(B, NEG)


# speedup vs baseline: 1.4570x; 1.2846x over previous
import functools

import jax
import jax.numpy as jnp
from jax import lax
from jax.experimental import pallas as pl
from jax.experimental.pallas import tpu as pltpu
from jax.experimental.pallas import tpu_sc as plsc

VOCAB = 1000000
D = 64
B = 16384
CTX = 20
NEG = 20
NR = D // 16

NC = 2
NS = 16
NW = NC * NS
BPW = B // NW
CB = 16
NCHUNK = BPW // CB
IDS_PER_CHUNK = CB * CTX
QROWS = 80
NQ = IDS_PER_CHUNK // QROWS
PSTRIDE = 17
KSLOT = CB * PSTRIDE
NK = NEG + 1


def _sc_body(ctx_ids2, neg_ids2, tgt_ids2, in_embed, out_embed,
             pos_out, neg_out,
             ctx_idx, neg_idx, tgt_idx, rows, pbuf, pos_stage, neg_stage,
             rowsem, outsem):
    c = lax.axis_index("c")
    s = lax.axis_index("s")
    wid = s * NC + c
    iota = lax.iota(jnp.int32, 16)
    iota17 = iota * PSTRIDE
    iota20 = iota * NEG

    pltpu.sync_copy(ctx_ids2.at[pl.ds(wid * (BPW * CTX // QROWS),
                                      BPW * CTX // QROWS)], ctx_idx)
    pltpu.sync_copy(neg_ids2.at[pl.ds(wid * (BPW * CTX // QROWS),
                                      BPW * CTX // QROWS)], neg_idx)
    pltpu.sync_copy(tgt_ids2.at[pl.ds(wid * NCHUNK, NCHUNK)], tgt_idx)

    def issue_rows(i):
        p = lax.rem(i, 2)
        r0 = i * NQ
        for q in range(NQ):
            pltpu.async_copy(
                in_embed.at[ctx_idx.at[r0 + q]],
                rows.at[p, pl.ds(q * QROWS, QROWS)], rowsem.at[p])
            pltpu.async_copy(
                out_embed.at[neg_idx.at[r0 + q]],
                rows.at[p, pl.ds(IDS_PER_CHUNK + q * QROWS, QROWS)],
                rowsem.at[p])
        pltpu.async_copy(out_embed.at[tgt_idx.at[i]],
                         rows.at[p, pl.ds(2 * IDS_PER_CHUNK, CB)],
                         rowsem.at[p])

    def drain_rows(i):
        p = lax.rem(i, 2)
        for q in range(NQ):
            pltpu.make_async_copy(
                in_embed.at[ctx_idx.at[0]],
                rows.at[p, pl.ds(q * QROWS, QROWS)], rowsem.at[p]).wait()
            pltpu.make_async_copy(
                out_embed.at[neg_idx.at[0]],
                rows.at[p, pl.ds(IDS_PER_CHUNK + q * QROWS, QROWS)],
                rowsem.at[p]).wait()
        pltpu.make_async_copy(out_embed.at[tgt_idx.at[0]],
                              rows.at[p, pl.ds(2 * IDS_PER_CHUNK, CB)],
                              rowsem.at[p]).wait()

    issue_rows(0)

    def chunk_body(i, carry_unused):
        p = lax.rem(i, 2)
        b0 = wid * BPW + i * CB

        @pl.when(i < NCHUNK - 1)
        def _():
            issue_rows(i + 1)

        drain_rows(i)

        def b_body(b, carry_unused2):
            base = b * CTX
            vc = [rows[p, base, pl.ds(r * 16, 16)] for r in range(NR)]
            for j in range(1, CTX):
                for r in range(NR):
                    vc[r] = vc[r] + rows[p, base + j, pl.ds(r * 16, 16)]
            inv = jnp.float32(1.0 / CTX)
            vc = [v * inv for v in vc]

            def dot_partial(row):
                t = [vc[r] * rows[p, row, pl.ds(r * 16, 16)]
                     for r in range(NR)]
                return (t[0] + t[1]) + (t[2] + t[3])

            slot = b * PSTRIDE
            pbuf[pl.ds(NEG * KSLOT + slot, 16)] = dot_partial(
                2 * IDS_PER_CHUNK + b)
            for k in range(NEG):
                pbuf[pl.ds(k * KSLOT + slot, 16)] = dot_partial(
                    IDS_PER_CHUNK + base + k)
            return carry_unused2

        lax.fori_loop(0, CB, b_body, 0, unroll=False)

        @pl.when(i >= 2)
        def _():
            pltpu.make_async_copy(pos_stage.at[p],
                                  pos_out.at[pl.ds(0, CB)], outsem.at[p]
                                  ).wait()
            pltpu.make_async_copy(neg_stage.at[p],
                                  neg_out.at[pl.ds(0, CB * NEG)],
                                  outsem.at[p]).wait()

        def treduce(k):
            vals = [plsc.load_gather(pbuf, [iota17 + (k * KSLOT + j)])
                    for j in range(16)]
            while len(vals) > 1:
                vals = [vals[2 * m] + vals[2 * m + 1]
                        for m in range(len(vals) // 2)]
            return vals[0]

        pos_stage[p] = treduce(NEG)
        for k in range(NEG):
            plsc.store_scatter(neg_stage.at[p], [iota20 + k], treduce(k))

        pltpu.async_copy(pos_stage.at[p], pos_out.at[pl.ds(b0, CB)],
                         outsem.at[p])
        pltpu.async_copy(neg_stage.at[p],
                         neg_out.at[pl.ds(b0 * NEG, CB * NEG)],
                         outsem.at[p])
        return carry_unused

    lax.fori_loop(0, NCHUNK, chunk_body, 0)

    for p in range(2):
        pltpu.make_async_copy(pos_stage.at[p], pos_out.at[pl.ds(0, CB)],
                              outsem.at[p]).wait()
        pltpu.make_async_copy(neg_stage.at[p],
                              neg_out.at[pl.ds(0, CB * NEG)],
                              outsem.at[p]).wait()


def _tp_body(src_ref, dst_ref):
    x = src_ref[...]
    r = jax.lax.broadcasted_iota(jnp.int32, (D, D), 0)
    c = jax.lax.broadcasted_iota(jnp.int32, (D, D), 1)
    iden = (r == c).astype(jnp.float32)
    dst_ref[...] = jax.lax.dot_general(
        x, iden, (((0,), (0,)), ((), ())),
        preferred_element_type=jnp.float32,
        precision=jax.lax.Precision.DEFAULT)


_TP_PARAMS = pltpu.CompilerParams(fuse_transposed_lhs_in_matmul=True)


def _transpose_table(table_t):
    s = 8192
    return pl.pallas_call(
        _tp_body,
        grid=((VOCAB + s - 1) // s,),
        in_specs=[pl.BlockSpec((D, s), lambda i: (0, i))],
        out_specs=pl.BlockSpec((s, D), lambda i: (i, 0)),
        out_shape=jax.ShapeDtypeStruct((VOCAB, D), jnp.float32),
        compiler_params=_TP_PARAMS,
    )(table_t)


@jax.jit
def _sc_call(ctx_ids2, neg_ids2, tgt_ids2, in_embed, out_embed):
    mesh = plsc.VectorSubcoreMesh(core_axis_name="c", subcore_axis_name="s")
    f = functools.partial(
        pl.kernel,
        out_type=[
            jax.ShapeDtypeStruct((B,), jnp.float32),
            jax.ShapeDtypeStruct((B * NEG,), jnp.float32),
        ],
        mesh=mesh,
        compiler_params=pltpu.CompilerParams(
            needs_layout_passes=False, use_tc_tiling_on_sc=False),
        scratch_types=[
            pltpu.VMEM((BPW * CTX // QROWS, QROWS), jnp.int32),
            pltpu.VMEM((BPW * CTX // QROWS, QROWS), jnp.int32),
            pltpu.VMEM((NCHUNK, CB), jnp.int32),
            pltpu.VMEM((2, 2 * IDS_PER_CHUNK + CB, D), jnp.float32),
            pltpu.VMEM((NK * KSLOT,), jnp.float32),
            pltpu.VMEM((2, CB), jnp.float32),
            pltpu.VMEM((2, CB * NEG), jnp.float32),
            pltpu.SemaphoreType.DMA((2,)),
            pltpu.SemaphoreType.DMA((2,)),
        ],
    )(_sc_body)
    return f(ctx_ids2, neg_ids2, tgt_ids2, in_embed, out_embed)


def kernel(context_ids, target_ids, neg_ids, in_embed, out_embed):
    ctx2 = jnp.asarray(context_ids, jnp.int32).reshape(B * CTX // QROWS, QROWS)
    neg2 = jnp.asarray(neg_ids, jnp.int32).reshape(B * NEG // QROWS, QROWS)
    tgt2 = jnp.asarray(target_ids, jnp.int32).reshape(B // CB, CB)
    in2 = _transpose_table(in_embed.T)
    out2 = _transpose_table(out_embed.T)
    pos, neg_flat = _sc_call(ctx2, neg2, tgt2, in2, out2)
    return pos, neg_flat.reshape(B, NEG)


# speedup vs baseline: 1.5068x; 1.0341x over previous
import functools

import jax
import jax.numpy as jnp
from jax import lax
from jax.experimental import pallas as pl
from jax.experimental.pallas import tpu as pltpu
from jax.experimental.pallas import tpu_sc as plsc

VOCAB = 1000000
D = 64
B = 16384
CTX = 20
NEG = 20
NR = D // 16

NC = 2
NS = 16
NW = NC * NS
BPW = B // NW
CB = 16
NCHUNK = BPW // CB
IDS_PER_CHUNK = CB * CTX
QROWS = 80
NQ = IDS_PER_CHUNK // QROWS
PSTRIDE = 17
KSLOT = CB * PSTRIDE
NK = NEG + 1


def _sc_body(ctx_ids2, neg_ids2, tgt_ids2, in_embed, out_embed,
             pos_out, neg_out,
             ctx_idx, neg_idx, tgt_idx, rows, pbuf, pos_stage, neg_stage,
             rowsem, outsem):
    c = lax.axis_index("c")
    s = lax.axis_index("s")
    wid = s * NC + c
    iota = lax.iota(jnp.int32, 16)
    iota17 = iota * PSTRIDE
    iota20 = iota * NEG

    pltpu.sync_copy(ctx_ids2.at[pl.ds(wid * (BPW * CTX // QROWS),
                                      BPW * CTX // QROWS)], ctx_idx)
    pltpu.sync_copy(neg_ids2.at[pl.ds(wid * (BPW * CTX // QROWS),
                                      BPW * CTX // QROWS)], neg_idx)
    pltpu.sync_copy(tgt_ids2.at[pl.ds(wid * NCHUNK, NCHUNK)], tgt_idx)

    def issue_rows(i):
        p = lax.rem(i, 2)
        r0 = i * NQ
        for q in range(NQ):
            pltpu.async_copy(
                in_embed.at[ctx_idx.at[r0 + q]],
                rows.at[p, pl.ds(q * QROWS, QROWS)], rowsem.at[p])
            pltpu.async_copy(
                out_embed.at[neg_idx.at[r0 + q]],
                rows.at[p, pl.ds(IDS_PER_CHUNK + q * QROWS, QROWS)],
                rowsem.at[p])
        pltpu.async_copy(out_embed.at[tgt_idx.at[i]],
                         rows.at[p, pl.ds(2 * IDS_PER_CHUNK, CB)],
                         rowsem.at[p])

    def drain_rows(i):
        p = lax.rem(i, 2)
        for q in range(NQ):
            pltpu.make_async_copy(
                in_embed.at[ctx_idx.at[0]],
                rows.at[p, pl.ds(q * QROWS, QROWS)], rowsem.at[p]).wait()
            pltpu.make_async_copy(
                out_embed.at[neg_idx.at[0]],
                rows.at[p, pl.ds(IDS_PER_CHUNK + q * QROWS, QROWS)],
                rowsem.at[p]).wait()
        pltpu.make_async_copy(out_embed.at[tgt_idx.at[0]],
                              rows.at[p, pl.ds(2 * IDS_PER_CHUNK, CB)],
                              rowsem.at[p]).wait()

    issue_rows(0)

    def chunk_body(i, carry_unused):
        p = lax.rem(i, 2)
        b0 = wid * BPW + i * CB

        @pl.when(i < NCHUNK - 1)
        def _():
            issue_rows(i + 1)

        drain_rows(i)

        def b_body(b, carry_unused2):
            base = b * CTX
            vc = [rows[p, base, pl.ds(r * 16, 16)] for r in range(NR)]
            for j in range(1, CTX):
                for r in range(NR):
                    vc[r] = vc[r] + rows[p, base + j, pl.ds(r * 16, 16)]
            inv = jnp.float32(1.0 / CTX)
            vc = [v * inv for v in vc]

            def dot_partial(row):
                t = [vc[r] * rows[p, row, pl.ds(r * 16, 16)]
                     for r in range(NR)]
                return (t[0] + t[1]) + (t[2] + t[3])

            slot = b * PSTRIDE
            pbuf[pl.ds(NEG * KSLOT + slot, 16)] = dot_partial(
                2 * IDS_PER_CHUNK + b)
            for k in range(NEG):
                pbuf[pl.ds(k * KSLOT + slot, 16)] = dot_partial(
                    IDS_PER_CHUNK + base + k)
            return carry_unused2

        lax.fori_loop(0, CB, b_body, 0, unroll=False)

        @pl.when(i >= 2)
        def _():
            pltpu.make_async_copy(pos_stage.at[p],
                                  pos_out.at[pl.ds(0, CB)], outsem.at[p]
                                  ).wait()
            pltpu.make_async_copy(neg_stage.at[p],
                                  neg_out.at[pl.ds(0, CB * NEG)],
                                  outsem.at[p]).wait()

        def treduce(k):
            vals = [plsc.load_gather(pbuf, [iota17 + (k * KSLOT + j)])
                    for j in range(16)]
            while len(vals) > 1:
                vals = [vals[2 * m] + vals[2 * m + 1]
                        for m in range(len(vals) // 2)]
            return vals[0]

        pos_stage[p] = treduce(NEG)
        for k in range(NEG):
            plsc.store_scatter(neg_stage.at[p], [iota20 + k], treduce(k))

        pltpu.async_copy(pos_stage.at[p], pos_out.at[pl.ds(b0, CB)],
                         outsem.at[p])
        pltpu.async_copy(neg_stage.at[p],
                         neg_out.at[pl.ds(b0 * NEG, CB * NEG)],
                         outsem.at[p])
        return carry_unused

    lax.fori_loop(0, NCHUNK, chunk_body, 0)

    for p in range(2):
        pltpu.make_async_copy(pos_stage.at[p], pos_out.at[pl.ds(0, CB)],
                              outsem.at[p]).wait()
        pltpu.make_async_copy(neg_stage.at[p],
                              neg_out.at[pl.ds(0, CB * NEG)],
                              outsem.at[p]).wait()


def _tp_body(src_ref, dst_ref):
    x = src_ref[...]
    r = jax.lax.broadcasted_iota(jnp.int32, (D, D), 0)
    c = jax.lax.broadcasted_iota(jnp.int32, (D, D), 1)
    iden = (r == c).astype(jnp.float32)
    dst_ref[...] = jax.lax.dot_general(
        x, iden, (((0,), (0,)), ((), ())),
        preferred_element_type=jnp.float32,
        precision=jax.lax.Precision.DEFAULT)


_TP_PARAMS = pltpu.CompilerParams(fuse_transposed_lhs_in_matmul=True)


def _transpose_table(table_t):
    s = 16384
    return pl.pallas_call(
        _tp_body,
        grid=((VOCAB + s - 1) // s,),
        in_specs=[pl.BlockSpec((D, s), lambda i: (0, i))],
        out_specs=pl.BlockSpec((s, D), lambda i: (i, 0)),
        out_shape=jax.ShapeDtypeStruct((VOCAB, D), jnp.float32),
        compiler_params=_TP_PARAMS,
    )(table_t)


@jax.jit
def _sc_call(ctx_ids2, neg_ids2, tgt_ids2, in_embed, out_embed):
    mesh = plsc.VectorSubcoreMesh(core_axis_name="c", subcore_axis_name="s")
    f = functools.partial(
        pl.kernel,
        out_type=[
            jax.ShapeDtypeStruct((B,), jnp.float32),
            jax.ShapeDtypeStruct((B * NEG,), jnp.float32),
        ],
        mesh=mesh,
        compiler_params=pltpu.CompilerParams(
            needs_layout_passes=False, use_tc_tiling_on_sc=False),
        scratch_types=[
            pltpu.VMEM((BPW * CTX // QROWS, QROWS), jnp.int32),
            pltpu.VMEM((BPW * CTX // QROWS, QROWS), jnp.int32),
            pltpu.VMEM((NCHUNK, CB), jnp.int32),
            pltpu.VMEM((2, 2 * IDS_PER_CHUNK + CB, D), jnp.float32),
            pltpu.VMEM((NK * KSLOT,), jnp.float32),
            pltpu.VMEM((2, CB), jnp.float32),
            pltpu.VMEM((2, CB * NEG), jnp.float32),
            pltpu.SemaphoreType.DMA((2,)),
            pltpu.SemaphoreType.DMA((2,)),
        ],
    )(_sc_body)
    return f(ctx_ids2, neg_ids2, tgt_ids2, in_embed, out_embed)


def kernel(context_ids, target_ids, neg_ids, in_embed, out_embed):
    ctx2 = jnp.asarray(context_ids, jnp.int32).reshape(B * CTX // QROWS, QROWS)
    neg2 = jnp.asarray(neg_ids, jnp.int32).reshape(B * NEG // QROWS, QROWS)
    tgt2 = jnp.asarray(target_ids, jnp.int32).reshape(B // CB, CB)
    in2 = _transpose_table(in_embed.T)
    out2 = _transpose_table(out_embed.T)
    pos, neg_flat = _sc_call(ctx2, neg2, tgt2, in2, out2)
    return pos, neg_flat.reshape(B, NEG)


# speedup vs baseline: 1.5767x; 1.0464x over previous
import functools

import jax
import jax.numpy as jnp
from jax import lax
from jax.experimental import pallas as pl
from jax.experimental.pallas import tpu as pltpu
from jax.experimental.pallas import tpu_sc as plsc

VOCAB = 1000000
D = 64
B = 16384
CTX = 20
NEG = 20
NR = D // 16

NC = 2
NS = 16
NW = NC * NS
BPW = B // NW
CB = 16
NCHUNK = BPW // CB
IDS_PER_CHUNK = CB * CTX
QROWS = 80
NQ = IDS_PER_CHUNK // QROWS
PSTRIDE = 17
KSLOT = CB * PSTRIDE
NK = NEG + 1


def _sc_body(ctx_ids2, neg_ids2, tgt_ids2, in_embed, out_embed,
             pos_out, neg_out,
             ctx_idx, neg_idx, tgt_idx, rows, pbuf, pos_stage, neg_stage,
             rowsem, outsem):
    c = lax.axis_index("c")
    s = lax.axis_index("s")
    wid = s * NC + c
    iota = lax.iota(jnp.int32, 16)
    iota17 = iota * PSTRIDE
    iota20 = iota * NEG

    pltpu.sync_copy(ctx_ids2.at[pl.ds(wid * (BPW * CTX // QROWS),
                                      BPW * CTX // QROWS)], ctx_idx)
    pltpu.sync_copy(neg_ids2.at[pl.ds(wid * (BPW * CTX // QROWS),
                                      BPW * CTX // QROWS)], neg_idx)
    pltpu.sync_copy(tgt_ids2.at[pl.ds(wid * NCHUNK, NCHUNK)], tgt_idx)

    def issue_rows(i):
        p = lax.rem(i, 2)
        r0 = i * NQ
        for q in range(NQ):
            pltpu.async_copy(
                in_embed.at[ctx_idx.at[r0 + q]],
                rows.at[p, pl.ds(q * QROWS, QROWS)], rowsem.at[p])
            pltpu.async_copy(
                out_embed.at[neg_idx.at[r0 + q]],
                rows.at[p, pl.ds(IDS_PER_CHUNK + q * QROWS, QROWS)],
                rowsem.at[p])
        pltpu.async_copy(out_embed.at[tgt_idx.at[i]],
                         rows.at[p, pl.ds(2 * IDS_PER_CHUNK, CB)],
                         rowsem.at[p])

    def drain_rows(i):
        p = lax.rem(i, 2)
        for q in range(NQ):
            pltpu.make_async_copy(
                in_embed.at[ctx_idx.at[0]],
                rows.at[p, pl.ds(q * QROWS, QROWS)], rowsem.at[p]).wait()
            pltpu.make_async_copy(
                out_embed.at[neg_idx.at[0]],
                rows.at[p, pl.ds(IDS_PER_CHUNK + q * QROWS, QROWS)],
                rowsem.at[p]).wait()
        pltpu.make_async_copy(out_embed.at[tgt_idx.at[0]],
                              rows.at[p, pl.ds(2 * IDS_PER_CHUNK, CB)],
                              rowsem.at[p]).wait()

    issue_rows(0)

    def chunk_body(i, carry_unused):
        p = lax.rem(i, 2)
        b0 = wid * BPW + i * CB

        @pl.when(i < NCHUNK - 1)
        def _():
            issue_rows(i + 1)

        drain_rows(i)

        def b_body(b, carry_unused2):
            base = b * CTX
            vc = [rows[p, base, pl.ds(r * 16, 16)] for r in range(NR)]
            for j in range(1, CTX):
                for r in range(NR):
                    vc[r] = vc[r] + rows[p, base + j, pl.ds(r * 16, 16)]
            inv = jnp.float32(1.0 / CTX)
            vc = [v * inv for v in vc]

            def dot_partial(row):
                t = [vc[r] * rows[p, row, pl.ds(r * 16, 16)]
                     for r in range(NR)]
                return (t[0] + t[1]) + (t[2] + t[3])

            slot = b * PSTRIDE
            pbuf[pl.ds(NEG * KSLOT + slot, 16)] = dot_partial(
                2 * IDS_PER_CHUNK + b)
            for k in range(NEG):
                pbuf[pl.ds(k * KSLOT + slot, 16)] = dot_partial(
                    IDS_PER_CHUNK + base + k)
            return carry_unused2

        lax.fori_loop(0, CB, b_body, 0, unroll=False)

        @pl.when(i >= 2)
        def _():
            pltpu.make_async_copy(pos_stage.at[p],
                                  pos_out.at[pl.ds(0, CB)], outsem.at[p]
                                  ).wait()
            pltpu.make_async_copy(neg_stage.at[p],
                                  neg_out.at[pl.ds(0, CB * NEG)],
                                  outsem.at[p]).wait()

        def treduce(k):
            vals = [plsc.load_gather(pbuf, [iota17 + (k * KSLOT + j)])
                    for j in range(16)]
            while len(vals) > 1:
                vals = [vals[2 * m] + vals[2 * m + 1]
                        for m in range(len(vals) // 2)]
            return vals[0]

        pos_stage[p] = treduce(NEG)
        for k in range(NEG):
            plsc.store_scatter(neg_stage.at[p], [iota20 + k], treduce(k))

        pltpu.async_copy(pos_stage.at[p], pos_out.at[pl.ds(b0, CB)],
                         outsem.at[p])
        pltpu.async_copy(neg_stage.at[p],
                         neg_out.at[pl.ds(b0 * NEG, CB * NEG)],
                         outsem.at[p])
        return carry_unused

    lax.fori_loop(0, NCHUNK, chunk_body, 0)

    for p in range(2):
        pltpu.make_async_copy(pos_stage.at[p], pos_out.at[pl.ds(0, CB)],
                              outsem.at[p]).wait()
        pltpu.make_async_copy(neg_stage.at[p],
                              neg_out.at[pl.ds(0, CB * NEG)],
                              outsem.at[p]).wait()


def _tp_body(src_ref, dst_ref):
    x = src_ref[...]
    r = jax.lax.broadcasted_iota(jnp.int32, (D, D), 0)
    c = jax.lax.broadcasted_iota(jnp.int32, (D, D), 1)
    iden = (r == c).astype(jnp.float32)
    dst_ref[...] = jax.lax.dot_general(
        x, iden, (((0,), (0,)), ((), ())),
        preferred_element_type=jnp.float32,
        precision=jax.lax.Precision.DEFAULT)


_TP_PARAMS = pltpu.CompilerParams(fuse_transposed_lhs_in_matmul=True)


def _transpose_table(table_t):
    s = 16384
    return pl.pallas_call(
        _tp_body,
        grid=((VOCAB + s - 1) // s,),
        in_specs=[pl.BlockSpec((D, s), lambda i: (0, i))],
        out_specs=pl.BlockSpec((s, D), lambda i: (i, 0)),
        out_shape=jax.ShapeDtypeStruct((VOCAB, D), jnp.float32),
        compiler_params=_TP_PARAMS,
    )(table_t)


@jax.jit
def _sc_call(ctx_ids2, neg_ids2, tgt_ids2, in_embed, out_embed):
    mesh = plsc.VectorSubcoreMesh(core_axis_name="c", subcore_axis_name="s")
    f = functools.partial(
        pl.kernel,
        out_type=[
            jax.ShapeDtypeStruct((B,), jnp.float32),
            jax.ShapeDtypeStruct((B * NEG,), jnp.float32),
        ],
        mesh=mesh,
        compiler_params=pltpu.CompilerParams(
            needs_layout_passes=False, use_tc_tiling_on_sc=False),
        scratch_types=[
            pltpu.VMEM((BPW * CTX // QROWS, QROWS), jnp.int32),
            pltpu.VMEM((BPW * CTX // QROWS, QROWS), jnp.int32),
            pltpu.VMEM((NCHUNK, CB), jnp.int32),
            pltpu.VMEM((2, 2 * IDS_PER_CHUNK + CB, D), jnp.float32),
            pltpu.VMEM((NK * KSLOT,), jnp.float32),
            pltpu.VMEM((2, CB), jnp.float32),
            pltpu.VMEM((2, CB * NEG), jnp.float32),
            pltpu.SemaphoreType.DMA((2,)),
            pltpu.SemaphoreType.DMA((2,)),
        ],
    )(_sc_body)
    return f(ctx_ids2, neg_ids2, tgt_ids2, in_embed, out_embed)


def kernel(context_ids, target_ids, neg_ids, in_embed, out_embed):
    ctx2 = jnp.asarray(context_ids, jnp.int32).reshape(B * CTX // QROWS, QROWS)
    neg2 = jnp.asarray(neg_ids, jnp.int32).reshape(B * NEG // QROWS, QROWS)
    tgt2 = jnp.asarray(target_ids, jnp.int32).reshape(B // CB, CB)
    in2 = _transpose_table(in_embed.T)
    pos, neg_flat = _sc_call(ctx2, neg2, tgt2, in2, out_embed)
    return pos, neg_flat.reshape(B, NEG)


# speedup vs baseline: 3.3608x; 2.1316x over previous
import functools

import jax
import jax.numpy as jnp
from jax import lax
from jax.experimental import pallas as pl
from jax.experimental.pallas import tpu as pltpu
from jax.experimental.pallas import tpu_sc as plsc

VOCAB = 1000000
D = 64
B = 16384
CTX = 20
NEG = 20
NR = D // 16

NC = 2
NS = 16
NW = NC * NS
BPW = B // NW
CB = 16
NCHUNK = BPW // CB
IDS_PER_CHUNK = CB * CTX
QROWS = 80
NQ = IDS_PER_CHUNK // QROWS
PSTRIDE = 17
KSLOT = CB * PSTRIDE
NK = NEG + 1


def _sc_body(ctx_ids2, neg_ids2, tgt_ids2, in_embed, out_embed,
             pos_out, neg_out,
             ctx_idx, neg_idx, tgt_idx, rows, pbuf, pos_stage, neg_stage,
             rowsem, outsem):
    c = lax.axis_index("c")
    s = lax.axis_index("s")
    wid = s * NC + c
    iota = lax.iota(jnp.int32, 16)
    iota17 = iota * PSTRIDE
    iota20 = iota * NEG

    pltpu.sync_copy(ctx_ids2.at[pl.ds(wid * (BPW * CTX // QROWS),
                                      BPW * CTX // QROWS)], ctx_idx)
    pltpu.sync_copy(neg_ids2.at[pl.ds(wid * (BPW * CTX // QROWS),
                                      BPW * CTX // QROWS)], neg_idx)
    pltpu.sync_copy(tgt_ids2.at[pl.ds(wid * NCHUNK, NCHUNK)], tgt_idx)

    def issue_rows(i):
        p = lax.rem(i, 2)
        r0 = i * NQ
        for q in range(NQ):
            pltpu.async_copy(
                in_embed.at[ctx_idx.at[r0 + q]],
                rows.at[p, pl.ds(q * QROWS, QROWS)], rowsem.at[p])
            pltpu.async_copy(
                out_embed.at[neg_idx.at[r0 + q]],
                rows.at[p, pl.ds(IDS_PER_CHUNK + q * QROWS, QROWS)],
                rowsem.at[p])
        pltpu.async_copy(out_embed.at[tgt_idx.at[i]],
                         rows.at[p, pl.ds(2 * IDS_PER_CHUNK, CB)],
                         rowsem.at[p])

    def drain_rows(i):
        p = lax.rem(i, 2)
        for q in range(NQ):
            pltpu.make_async_copy(
                in_embed.at[ctx_idx.at[0]],
                rows.at[p, pl.ds(q * QROWS, QROWS)], rowsem.at[p]).wait()
            pltpu.make_async_copy(
                out_embed.at[neg_idx.at[0]],
                rows.at[p, pl.ds(IDS_PER_CHUNK + q * QROWS, QROWS)],
                rowsem.at[p]).wait()
        pltpu.make_async_copy(out_embed.at[tgt_idx.at[0]],
                              rows.at[p, pl.ds(2 * IDS_PER_CHUNK, CB)],
                              rowsem.at[p]).wait()

    issue_rows(0)

    def chunk_body(i, carry_unused):
        p = lax.rem(i, 2)
        b0 = wid * BPW + i * CB

        @pl.when(i < NCHUNK - 1)
        def _():
            issue_rows(i + 1)

        drain_rows(i)

        def b_body(b, carry_unused2):
            base = b * CTX
            vc = [rows[p, base, pl.ds(r * 16, 16)] for r in range(NR)]
            for j in range(1, CTX):
                for r in range(NR):
                    vc[r] = vc[r] + rows[p, base + j, pl.ds(r * 16, 16)]
            inv = jnp.float32(1.0 / CTX)
            vc = [v * inv for v in vc]

            def dot_partial(row):
                t = [vc[r] * rows[p, row, pl.ds(r * 16, 16)]
                     for r in range(NR)]
                return (t[0] + t[1]) + (t[2] + t[3])

            slot = b * PSTRIDE
            pbuf[pl.ds(NEG * KSLOT + slot, 16)] = dot_partial(
                2 * IDS_PER_CHUNK + b)
            for k in range(NEG):
                pbuf[pl.ds(k * KSLOT + slot, 16)] = dot_partial(
                    IDS_PER_CHUNK + base + k)
            return carry_unused2

        lax.fori_loop(0, CB, b_body, 0, unroll=False)

        @pl.when(i >= 2)
        def _():
            pltpu.make_async_copy(pos_stage.at[p],
                                  pos_out.at[pl.ds(0, CB)], outsem.at[p]
                                  ).wait()
            pltpu.make_async_copy(neg_stage.at[p],
                                  neg_out.at[pl.ds(0, CB * NEG)],
                                  outsem.at[p]).wait()

        def treduce(k):
            vals = [plsc.load_gather(pbuf, [iota17 + (k * KSLOT + j)])
                    for j in range(16)]
            while len(vals) > 1:
                vals = [vals[2 * m] + vals[2 * m + 1]
                        for m in range(len(vals) // 2)]
            return vals[0]

        pos_stage[p] = treduce(NEG)
        for k in range(NEG):
            plsc.store_scatter(neg_stage.at[p], [iota20 + k], treduce(k))

        pltpu.async_copy(pos_stage.at[p], pos_out.at[pl.ds(b0, CB)],
                         outsem.at[p])
        pltpu.async_copy(neg_stage.at[p],
                         neg_out.at[pl.ds(b0 * NEG, CB * NEG)],
                         outsem.at[p])
        return carry_unused

    lax.fori_loop(0, NCHUNK, chunk_body, 0)

    for p in range(2):
        pltpu.make_async_copy(pos_stage.at[p], pos_out.at[pl.ds(0, CB)],
                              outsem.at[p]).wait()
        pltpu.make_async_copy(neg_stage.at[p],
                              neg_out.at[pl.ds(0, CB * NEG)],
                              outsem.at[p]).wait()


TPS = 8192
NTPB = 62
VH = TPS * NTPB
VP = 2 * VH


def _tp_body(src_ref, dst_ref):
    r = jax.lax.broadcasted_iota(jnp.int32, (D, D), 0)
    c = jax.lax.broadcasted_iota(jnp.int32, (D, D), 1)
    iden = (r == c).astype(jnp.float32)

    def tp(x):
        return jax.lax.dot_general(
            x, iden, (((0,), (0,)), ((), ())),
            preferred_element_type=jnp.float32,
            precision=jax.lax.Precision.DEFAULT)
    x = src_ref[...]
    dst_ref[...] = jnp.concatenate(
        [tp(x[:, :TPS]), tp(x[:, TPS:])], axis=1)


_TP_PARAMS = pltpu.CompilerParams(fuse_transposed_lhs_in_matmul=True)


def _transpose_table(table_t):
    return pl.pallas_call(
        _tp_body,
        grid=(NTPB,),
        in_specs=[pl.BlockSpec((D, 2 * TPS), lambda i: (0, i))],
        out_specs=pl.BlockSpec((TPS, 2 * D), lambda i: (i, 0)),
        out_shape=jax.ShapeDtypeStruct((VH, 2 * D), jnp.float32),
        compiler_params=_TP_PARAMS,
    )(table_t)


@jax.jit
def _sc_call(ctx_ids2, neg_ids2, tgt_ids2, in_embed, out_embed):
    mesh = plsc.VectorSubcoreMesh(core_axis_name="c", subcore_axis_name="s")
    f = functools.partial(
        pl.kernel,
        out_type=[
            jax.ShapeDtypeStruct((B,), jnp.float32),
            jax.ShapeDtypeStruct((B * NEG,), jnp.float32),
        ],
        mesh=mesh,
        compiler_params=pltpu.CompilerParams(
            needs_layout_passes=False, use_tc_tiling_on_sc=False),
        scratch_types=[
            pltpu.VMEM((BPW * CTX // QROWS, QROWS), jnp.int32),
            pltpu.VMEM((BPW * CTX // QROWS, QROWS), jnp.int32),
            pltpu.VMEM((NCHUNK, CB), jnp.int32),
            pltpu.VMEM((2, 2 * IDS_PER_CHUNK + CB, D), jnp.float32),
            pltpu.VMEM((NK * KSLOT,), jnp.float32),
            pltpu.VMEM((2, CB), jnp.float32),
            pltpu.VMEM((2, CB * NEG), jnp.float32),
            pltpu.SemaphoreType.DMA((2,)),
            pltpu.SemaphoreType.DMA((2,)),
        ],
    )(_sc_body)
    return f(ctx_ids2, neg_ids2, tgt_ids2, in_embed, out_embed)


def _remap(ids):
    v = jnp.asarray(ids, jnp.int32)
    return ((v >> 14) << 14) + ((v & (TPS - 1)) << 1) + ((v >> 13) & 1)


def kernel(context_ids, target_ids, neg_ids, in_embed, out_embed):
    ctx2 = _remap(context_ids).reshape(B * CTX // QROWS, QROWS)
    neg2 = _remap(neg_ids).reshape(B * NEG // QROWS, QROWS)
    tgt2 = _remap(target_ids).reshape(B // CB, CB)
    in2 = _transpose_table(in_embed.T).reshape(VP, D)
    out2 = _transpose_table(out_embed.T).reshape(VP, D)
    pos, neg_flat = _sc_call(ctx2, neg2, tgt2, in2, out2)
    return pos, neg_flat.reshape(B, NEG)
